# baseline (speedup 1.0000x reference)
import functools

import jax
import jax.numpy as jnp
from jax import lax
from jax.experimental import pallas as pl
from jax.experimental.pallas import tpu as pltpu
from jax.experimental.pallas import tpu_sc as plsc

N_NODES = 10000
N_EDGES = 320000
D_FEAT = 128

NUM_CORES = 2
NUM_SUBCORES = 16
NUM_WORKERS = NUM_CORES * NUM_SUBCORES
EDGES_PER_WORKER = N_EDGES // NUM_WORKERS
CHUNK = 80
NBUF = 5
SUPER = CHUNK * NBUF
NUM_SUPER = EDGES_PER_WORKER // SUPER
L = 16


def _sc_kernel_body(src_hbm, tgt_hbm, xu_hbm, xi_hbm, out_hbm,
                    idx_u, idx_i, rows_u, rows_i, out_v, sem_u, sem_i, sem_x):
    wid = lax.axis_index("s") * NUM_CORES + lax.axis_index("c")
    base_w = wid * EDGES_PER_WORKER

    lane = lax.broadcasted_iota(jnp.int32, (L,), 0)
    perms = [lax.bitwise_xor(lane, jnp.int32(st)) for st in (8, 4, 2, 1)]

    def compute_chunk(base, b):
        def group_body(g, carry):
            accs = [None] * L
            for k in range(D_FEAT // L):
                for e in range(L):
                    row = g * L + e
                    u = rows_u.at[b][row, pl.ds(k * L, L)]
                    v = rows_i.at[b][row, pl.ds(k * L, L)]
                    uv = u * v
                    accs[e] = uv if k == 0 else accs[e] + uv
            res = jnp.zeros((L,), jnp.float32)
            for e in range(L):
                p = accs[e]
                for perm in perms:
                    p = p + p[perm]
                res = res + jnp.where(lane == e, p, 0.0)
            out_v[pl.ds(b * CHUNK + g * L, L)] = 1.0 / (1.0 + jnp.exp(-res))
            return carry

        lax.fori_loop(0, CHUNK // L, group_body, 0)

    @pl.loop(0, NUM_SUPER)
    def super_body(sc):
        base = sc * SUPER
        diu = pltpu.async_copy(src_hbm.at[pl.ds(base_w + base, SUPER)], idx_u, sem_x)
        dii = pltpu.async_copy(tgt_hbm.at[pl.ds(base_w + base, SUPER)], idx_i, sem_x)
        diu.wait()
        dii.wait()
        descs = []
        for b in range(NBUF):
            s = pl.ds(b * CHUNK, CHUNK)
            du = pltpu.async_copy(xu_hbm.at[idx_u.at[s]], rows_u.at[b], sem_u)
            di = pltpu.async_copy(xi_hbm.at[idx_i.at[s]], rows_i.at[b], sem_i)
            descs.append((du, di))
        for b in range(NBUF):
            du, di = descs[b]
            du.wait()
            di.wait()
            compute_chunk(base, b)
        pltpu.sync_copy(out_v, out_hbm.at[pl.ds(base_w + base, SUPER)])


@jax.jit
def _decode(x_user, x_item, src, tgt):
    mesh = plsc.VectorSubcoreMesh(
        core_axis_name="c", subcore_axis_name="s",
        num_cores=NUM_CORES, num_subcores=NUM_SUBCORES)
    run = pl.kernel(
        _sc_kernel_body,
        out_type=jax.ShapeDtypeStruct((N_EDGES,), jnp.float32),
        mesh=mesh,
        scratch_types=[
            pltpu.VMEM((SUPER,), jnp.int32),
            pltpu.VMEM((SUPER,), jnp.int32),
            pltpu.VMEM((NBUF, CHUNK, D_FEAT), jnp.float32),
            pltpu.VMEM((NBUF, CHUNK, D_FEAT), jnp.float32),
            pltpu.VMEM((SUPER,), jnp.float32),
            pltpu.SemaphoreType.DMA,
            pltpu.SemaphoreType.DMA,
            pltpu.SemaphoreType.DMA,
        ],
        compiler_params=pltpu.CompilerParams(needs_layout_passes=False),
    )
    return run(src, tgt, x_user, x_item)


def kernel(x_user, x_item, edge_index):
    src = edge_index[0].astype(jnp.int32)
    tgt = edge_index[1].astype(jnp.int32)
    return _decode(x_user, x_item, src, tgt)


# speedup vs baseline: 1.0514x; 1.0514x over previous
import functools

import jax
import jax.numpy as jnp
from jax import lax
from jax.experimental import pallas as pl
from jax.experimental.pallas import tpu as pltpu
from jax.experimental.pallas import tpu_sc as plsc

N_NODES = 10000
N_EDGES = 320000
D_FEAT = 128

NUM_CORES = 2
NUM_SUBCORES = 16
NUM_WORKERS = NUM_CORES * NUM_SUBCORES
EDGES_PER_WORKER = N_EDGES // NUM_WORKERS
CHUNK = 80
NSUB = 5
NBUF = 2
SUPER = CHUNK * NSUB
NUM_SUPER = EDGES_PER_WORKER // SUPER
L = 16


def _sc_kernel_body(src_hbm, tgt_hbm, xu_hbm, xi_hbm, out_hbm,
                    idx_u, idx_i, rows_u, rows_i, out_v, sem_u, sem_i):
    wid = lax.axis_index("s") * NUM_CORES + lax.axis_index("c")
    base_w = wid * EDGES_PER_WORKER

    du = pltpu.async_copy(src_hbm.at[pl.ds(base_w, EDGES_PER_WORKER)], idx_u, sem_u)
    di = pltpu.async_copy(tgt_hbm.at[pl.ds(base_w, EDGES_PER_WORKER)], idx_i, sem_i)
    du.wait()
    di.wait()

    lane = lax.broadcasted_iota(jnp.int32, (L,), 0)
    perms = [lax.bitwise_xor(lane, jnp.int32(st)) for st in (8, 4, 2, 1)]

    def start_gather(base, j, b):
        s = pl.ds(base + j * CHUNK, CHUNK)
        du = pltpu.async_copy(xu_hbm.at[idx_u.at[s]], rows_u.at[b], sem_u)
        di = pltpu.async_copy(xi_hbm.at[idx_i.at[s]], rows_i.at[b], sem_i)
        return du, di

    def compute_chunk(base, j, b):
        def group_body(g, carry):
            res = jnp.zeros((L,), jnp.float32)
            for e in range(0, L, 2):
                rowa = g * L + e
                rowb = g * L + e + 1
                aa = None
                ab = None
                for k in range(D_FEAT // L):
                    ua = rows_u.at[b][rowa, pl.ds(k * L, L)]
                    va = rows_i.at[b][rowa, pl.ds(k * L, L)]
                    ub = rows_u.at[b][rowb, pl.ds(k * L, L)]
                    vb = rows_i.at[b][rowb, pl.ds(k * L, L)]
                    aa = ua * va if aa is None else aa + ua * va
                    ab = ub * vb if ab is None else ab + ub * vb
                pa, pb = aa, ab
                for perm in perms:
                    pa = pa + pa[perm]
                    pb = pb + pb[perm]
                res = res + jnp.where(lane == e, pa, 0.0)
                res = res + jnp.where(lane == e + 1, pb, 0.0)
            out_v[pl.ds(base + j * CHUNK + g * L, L)] = 1.0 / (1.0 + jnp.exp(-res))
            return carry

        lax.fori_loop(0, CHUNK // L, group_body, 0)

    @pl.loop(0, NUM_SUPER)
    def super_body(sc):
        base = sc * SUPER
        descs = {}
        for j in range(NBUF):
            descs[j] = start_gather(base, j, j)
        for j in range(NSUB):
            du, di = descs[j]
            du.wait()
            di.wait()
            compute_chunk(base, j, j % NBUF)
            if j + NBUF < NSUB:
                descs[j + NBUF] = start_gather(base, j + NBUF, j % NBUF)

    pltpu.sync_copy(out_v, out_hbm.at[pl.ds(base_w, EDGES_PER_WORKER)])


@jax.jit
def _decode(x_user, x_item, src, tgt):
    mesh = plsc.VectorSubcoreMesh(
        core_axis_name="c", subcore_axis_name="s",
        num_cores=NUM_CORES, num_subcores=NUM_SUBCORES)
    run = pl.kernel(
        _sc_kernel_body,
        out_type=jax.ShapeDtypeStruct((N_EDGES,), jnp.float32),
        mesh=mesh,
        scratch_types=[
            pltpu.VMEM((EDGES_PER_WORKER,), jnp.int32),
            pltpu.VMEM((EDGES_PER_WORKER,), jnp.int32),
            pltpu.VMEM((NBUF, CHUNK, D_FEAT), jnp.float32),
            pltpu.VMEM((NBUF, CHUNK, D_FEAT), jnp.float32),
            pltpu.VMEM((EDGES_PER_WORKER,), jnp.float32),
            pltpu.SemaphoreType.DMA,
            pltpu.SemaphoreType.DMA,
        ],
        compiler_params=pltpu.CompilerParams(needs_layout_passes=False),
    )
    return run(src, tgt, x_user, x_item)


def kernel(x_user, x_item, edge_index):
    src = edge_index[0].astype(jnp.int32)
    tgt = edge_index[1].astype(jnp.int32)
    return _decode(x_user, x_item, src, tgt)
